# baseline (speedup 1.0000x reference)
import jax
import jax.numpy as jnp
from jax.experimental import pallas as pl
from jax.experimental.pallas import tpu as pltpu

F32 = jnp.float32
L = 256
N_CAND = 64
K_NN = 9
G = 8
R = G * L

_OFFSETS = [(dy, dx) for dy in (-1, 0, 1) for dx in (-1, 0, 1)]


def _mm(a, b, prec=None):
    return jax.lax.dot_general(
        a, b, (((1,), (0,)), ((), ())),
        preferred_element_type=F32, precision=prec)


def _mm_rt(a, b, prec=None):
    return jax.lax.dot_general(
        a, b, (((1,), (1,)), ((), ())),
        preferred_element_type=F32, precision=prec)


def _roll_rows(x, s):
    s = s % x.shape[0]
    if s == 0:
        return x
    return jnp.concatenate([x[s:, :], x[:s, :]], axis=0)


def _convnn_layer(x_all, masks, sel, wc_t, wn_t, wp_t, bc, bn, bp, cdim, odim):
    pieces = []
    for j, (dy, dx) in enumerate(_OFFSETS):
        pieces.append(_roll_rows(x_all, dy * 16 + dx) * masks[j])
    xim = jnp.concatenate(pieces, axis=1)
    b1 = _mm(xim, wc_t) + bc

    cands, sims = [], []
    for g in range(G):
        xg = x_all[g * L:(g + 1) * L, :]
        cg = _mm(sel, xg)
        cands.append(cg)
        sims.append(_mm_rt(xg, cg))
    cand_all = jnp.concatenate(cands, axis=0)
    s = jnp.concatenate(sims, axis=0)

    iota_n = jax.lax.broadcasted_iota(jnp.int32, (R, N_CAND), 1)
    onehots = []
    for _ in range(K_NN):
        m = jnp.max(s, axis=1, keepdims=True)
        idx = jnp.min(jnp.where(s == m, iota_n, N_CAND), axis=1, keepdims=True)
        hit = iota_n == idx
        onehots.append(hit.astype(F32))
        s = jnp.where(hit, -jnp.inf, s)

    p_all = _mm(cand_all, wn_t)
    b2s = []
    for g in range(G):
        a_g = jnp.concatenate(
            [oh[g * L:(g + 1) * L, :] for oh in onehots], axis=1)
        p_g = jnp.concatenate(
            [p_all[g * N_CAND:(g + 1) * N_CAND, k * odim:(k + 1) * odim]
             for k in range(K_NN)], axis=0)
        b2s.append(_mm(a_g, p_g, jax.lax.Precision.HIGHEST))
    b2 = jnp.concatenate(b2s, axis=0) + bn

    cat = jnp.concatenate([b1, b2], axis=1)
    return jax.nn.relu(_mm(cat, wp_t) + bp)


def _conv_kernel(x_ref, si1_ref, si2_ref,
                 wc1_ref, wn1_ref, wp1_ref, bc1_ref, bn1_ref, bp1_ref,
                 wc2_ref, wn2_ref, wp2_ref, bc2_ref, bn2_ref, bp2_ref,
                 out_ref):
    r_iota = jax.lax.broadcasted_iota(jnp.int32, (R, 1), 0)
    l_pix = r_iota % L
    h, w = l_pix // 16, l_pix % 16
    masks = []
    for dy, dx in _OFFSETS:
        ok = (h + dy >= 0) & (h + dy <= 15) & (w + dx >= 0) & (w + dx <= 15)
        masks.append(ok.astype(F32))

    iota_l1 = jax.lax.broadcasted_iota(jnp.int32, (N_CAND, L), 1)
    sel1 = (si1_ref[...] == iota_l1).astype(F32)
    sel2 = (si2_ref[...] == iota_l1).astype(F32)

    x1 = x_ref[...]
    h1 = _convnn_layer(x1, masks, sel1, wc1_ref[...], wn1_ref[...],
                       wp1_ref[...], bc1_ref[...], bn1_ref[...], bp1_ref[...],
                       12, 16)
    h2 = _convnn_layer(h1, masks, sel2, wc2_ref[...], wn2_ref[...],
                       wp2_ref[...], bc2_ref[...], bn2_ref[...], bp2_ref[...],
                       64, 32)
    out_ref[...] = h2


def _fc1_kernel(h_ref, w_ref, b_ref, z_ref, acc_ref):
    k = pl.program_id(1)
    nk = pl.num_programs(1)

    @pl.when(k == 0)
    def _():
        acc_ref[...] = jnp.zeros_like(acc_ref)

    acc_ref[...] += _mm_rt(h_ref[...], w_ref[...])

    @pl.when(k == nk - 1)
    def _():
        z_ref[...] = jax.nn.relu(acc_ref[...] + b_ref[...])


def _fc2_kernel(z_ref, w_ref, b_ref, out_ref):
    out_ref[...] = _mm_rt(z_ref[...], w_ref[...]) + b_ref[...]


def kernel(x, W1c, b1c, W1n, b1n, W1p, b1p, W2c, b2c, W2n, b2n, W2p, b2p,
           Wfc1, bfc1, Wfc2, bfc2, sample_idx1, sample_idx2):
    B = x.shape[0]

    xu = x.reshape(B, 3, 16, 2, 16, 2).transpose(0, 1, 3, 5, 2, 4)
    xu = xu.reshape(B, 12, 256).transpose(0, 2, 1).reshape(B * L, 12)

    wc1_t = W1c.transpose(2, 3, 1, 0).reshape(9 * 12, 16)
    wc2_t = W2c.transpose(2, 3, 1, 0).reshape(9 * 64, 32)
    wn1_t = W1n.transpose(1, 2, 0).reshape(12, 9 * 16)
    wn2_t = W2n.transpose(1, 2, 0).reshape(64, 9 * 32)
    wp1_t = W1p.reshape(64, 32).transpose(1, 0)
    wp2_t = W2p.reshape(128, 64).transpose(1, 0)
    row = lambda v: v.reshape(1, -1)
    si1 = sample_idx1.reshape(N_CAND, 1)
    si2 = sample_idx2.reshape(N_CAND, 1)


    full = lambda shape: pl.BlockSpec(shape, lambda ii: (0, 0))
    h2 = pl.pallas_call(
        _conv_kernel,
        grid=(B // G,),
        in_specs=[
            pl.BlockSpec((R, 12), lambda ii: (ii, 0)),
            full((N_CAND, 1)), full((N_CAND, 1)),
            full(wc1_t.shape), full(wn1_t.shape), full(wp1_t.shape),
            full((1, 16)), full((1, 16)), full((1, 64)),
            full(wc2_t.shape), full(wn2_t.shape), full(wp2_t.shape),
            full((1, 32)), full((1, 32)), full((1, 128)),
        ],
        out_specs=pl.BlockSpec((R, 128), lambda ii: (ii, 0)),
        out_shape=jax.ShapeDtypeStruct((B * L, 128), F32),
        compiler_params=pltpu.CompilerParams(
            dimension_semantics=("parallel",)),
    )(xu, si1, si2, wc1_t, wn1_t, wp1_t, row(b1c), row(b1n), row(b1p),
      wc2_t, wn2_t, wp2_t, row(b2c), row(b2n), row(b2p))

    hflat = (h2.reshape(B, 16, 16, 32, 2, 2)
             .transpose(0, 3, 1, 4, 2, 5).reshape(B, 32768))

    KT, NT = 2048, 512
    z = pl.pallas_call(
        _fc1_kernel,
        grid=(1024 // NT, 32768 // KT),
        in_specs=[
            pl.BlockSpec((B, KT), lambda n, k: (0, k)),
            pl.BlockSpec((NT, KT), lambda n, k: (n, k)),
            pl.BlockSpec((1, NT), lambda n, k: (0, n)),
        ],
        out_specs=pl.BlockSpec((B, NT), lambda n, k: (0, n)),
        out_shape=jax.ShapeDtypeStruct((B, 1024), F32),
        scratch_shapes=[pltpu.VMEM((B, NT), F32)],
        compiler_params=pltpu.CompilerParams(
            dimension_semantics=("parallel", "arbitrary")),
    )(hflat, Wfc1, row(bfc1))

    return pl.pallas_call(
        _fc2_kernel,
        in_specs=[pl.BlockSpec(z.shape, lambda: (0, 0)),
                  pl.BlockSpec(Wfc2.shape, lambda: (0, 0)),
                  pl.BlockSpec((1, 10), lambda: (0, 0))],
        out_specs=pl.BlockSpec((B, 10), lambda: (0, 0)),
        out_shape=jax.ShapeDtypeStruct((B, 10), F32),
    )(z, Wfc2, row(bfc2))


# speedup vs baseline: 1.3368x; 1.3368x over previous
import jax
import jax.numpy as jnp
from jax.experimental import pallas as pl
from jax.experimental.pallas import tpu as pltpu

F32 = jnp.float32
L = 256
N_CAND = 64
K_NN = 9
G = 8
R = G * L

_OFFSETS = [(dy, dx) for dy in (-1, 0, 1) for dx in (-1, 0, 1)]


def _mm(a, b, prec=None):
    return jax.lax.dot_general(
        a, b, (((1,), (0,)), ((), ())),
        preferred_element_type=F32, precision=prec)


def _mm_rt(a, b, prec=None):
    return jax.lax.dot_general(
        a, b, (((1,), (1,)), ((), ())),
        preferred_element_type=F32, precision=prec)


def _roll_rows(x, s):
    s = s % x.shape[0]
    if s == 0:
        return x
    return jnp.concatenate([x[s:, :], x[:s, :]], axis=0)


def _convnn_layer(x_all, masks, sel, lt, wc_t, wn_t, wp_t, bc, bn, bp,
                  cdim, odim):
    pieces = []
    for j, (dy, dx) in enumerate(_OFFSETS):
        pieces.append(_roll_rows(x_all, dy * 16 + dx) * masks[j])
    xim = jnp.concatenate(pieces, axis=1)
    b1 = _mm(xim, wc_t) + bc

    cands, sims = [], []
    for g in range(G):
        xg = x_all[g * L:(g + 1) * L, :]
        cg = _mm(sel, xg)
        cands.append(cg)
        sims.append(_mm_rt(xg, cg))
    cand_all = jnp.concatenate(cands, axis=0)
    s = jnp.concatenate(sims, axis=0)

    onehots = []
    for _ in range(K_NN):
        m = jnp.max(s, axis=1, keepdims=True)
        hitb = s == m
        cnt = _mm(hitb.astype(F32), lt)
        ohb = hitb & (cnt < 0.5)
        onehots.append(ohb.astype(F32))
        s = jnp.where(ohb, -jnp.inf, s)

    p_all = _mm(cand_all, wn_t)
    b2s = []
    for g in range(G):
        a_g = jnp.concatenate(
            [oh[g * L:(g + 1) * L, :] for oh in onehots], axis=1)
        p_g = jnp.concatenate(
            [p_all[g * N_CAND:(g + 1) * N_CAND, k * odim:(k + 1) * odim]
             for k in range(K_NN)], axis=0)
        b2s.append(_mm(a_g, p_g, jax.lax.Precision.HIGHEST))
    b2 = jnp.concatenate(b2s, axis=0) + bn

    cat = jnp.concatenate([b1, b2], axis=1)
    return jax.nn.relu(_mm(cat, wp_t) + bp)


def _conv_kernel(x_ref, si1_ref, si2_ref,
                 wc1_ref, wn1_ref, wp1_ref, bc1_ref, bn1_ref, bp1_ref,
                 wc2_ref, wn2_ref, wp2_ref, bc2_ref, bn2_ref, bp2_ref,
                 out_ref):
    r_iota = jax.lax.broadcasted_iota(jnp.int32, (R, 1), 0)
    l_pix = r_iota % L
    h, w = l_pix // 16, l_pix % 16
    masks = []
    for dy, dx in _OFFSETS:
        ok = (h + dy >= 0) & (h + dy <= 15) & (w + dx >= 0) & (w + dx <= 15)
        masks.append(ok.astype(F32))

    iota_l1 = jax.lax.broadcasted_iota(jnp.int32, (N_CAND, L), 1)
    sel1 = (si1_ref[...] == iota_l1).astype(F32)
    sel2 = (si2_ref[...] == iota_l1).astype(F32)

    r_n = jax.lax.broadcasted_iota(jnp.int32, (N_CAND, N_CAND), 0)
    c_n = jax.lax.broadcasted_iota(jnp.int32, (N_CAND, N_CAND), 1)
    lt = (r_n < c_n).astype(F32)

    x1 = x_ref[...]
    h1 = _convnn_layer(x1, masks, sel1, lt, wc1_ref[...], wn1_ref[...],
                       wp1_ref[...], bc1_ref[...], bn1_ref[...], bp1_ref[...],
                       12, 16)
    h2 = _convnn_layer(h1, masks, sel2, lt, wc2_ref[...], wn2_ref[...],
                       wp2_ref[...], bc2_ref[...], bn2_ref[...], bp2_ref[...],
                       64, 32)
    out_ref[...] = h2


def _fc1_kernel(h_ref, w_ref, b_ref, z_ref, acc_ref):
    k = pl.program_id(1)
    nk = pl.num_programs(1)

    @pl.when(k == 0)
    def _():
        acc_ref[...] = jnp.zeros_like(acc_ref)

    acc_ref[...] += _mm(h_ref[...], w_ref[...])

    @pl.when(k == nk - 1)
    def _():
        z_ref[...] = jax.nn.relu(acc_ref[...] + b_ref[...])


def _fc2_kernel(z_ref, w_ref, b_ref, out_ref):
    out_ref[...] = _mm_rt(z_ref[...], w_ref[...]) + b_ref[...]


def kernel(x, W1c, b1c, W1n, b1n, W1p, b1p, W2c, b2c, W2n, b2n, W2p, b2p,
           Wfc1, bfc1, Wfc2, bfc2, sample_idx1, sample_idx2):
    B = x.shape[0]

    xu = x.reshape(B, 3, 16, 2, 16, 2).transpose(0, 1, 3, 5, 2, 4)
    xu = xu.reshape(B, 12, 256).transpose(0, 2, 1).reshape(B * L, 12)

    wc1_t = W1c.transpose(2, 3, 1, 0).reshape(9 * 12, 16)
    wc2_t = W2c.transpose(2, 3, 1, 0).reshape(9 * 64, 32)
    wn1_t = W1n.transpose(1, 2, 0).reshape(12, 9 * 16)
    wn2_t = W2n.transpose(1, 2, 0).reshape(64, 9 * 32)
    wp1_t = W1p.reshape(64, 32).transpose(1, 0)
    wp2_t = W2p.reshape(128, 64).transpose(1, 0)
    row = lambda v: v.reshape(1, -1)
    si1 = sample_idx1.reshape(N_CAND, 1)
    si2 = sample_idx2.reshape(N_CAND, 1)


    full = lambda shape: pl.BlockSpec(shape, lambda ii: (0, 0))
    h2 = pl.pallas_call(
        _conv_kernel,
        grid=(B // G,),
        in_specs=[
            pl.BlockSpec((R, 12), lambda ii: (ii, 0)),
            full((N_CAND, 1)), full((N_CAND, 1)),
            full(wc1_t.shape), full(wn1_t.shape), full(wp1_t.shape),
            full((1, 16)), full((1, 16)), full((1, 64)),
            full(wc2_t.shape), full(wn2_t.shape), full(wp2_t.shape),
            full((1, 32)), full((1, 32)), full((1, 128)),
        ],
        out_specs=pl.BlockSpec((R, 128), lambda ii: (ii, 0)),
        out_shape=jax.ShapeDtypeStruct((B * L, 128), F32),
        compiler_params=pltpu.CompilerParams(
            dimension_semantics=("parallel",)),
    )(xu, si1, si2, wc1_t, wn1_t, wp1_t, row(b1c), row(b1n), row(b1p),
      wc2_t, wn2_t, wp2_t, row(b2c), row(b2n), row(b2p))

    hflat = h2.reshape(B, 32768)

    w1r = (Wfc1.reshape(1024, 32, 16, 2, 16, 2)
           .transpose(2, 4, 1, 3, 5, 0).reshape(32768, 1024))

    KT, NT = 2048, 512
    z = pl.pallas_call(
        _fc1_kernel,
        grid=(1024 // NT, 32768 // KT),
        in_specs=[
            pl.BlockSpec((B, KT), lambda n, k: (0, k)),
            pl.BlockSpec((KT, NT), lambda n, k: (k, n)),
            pl.BlockSpec((1, NT), lambda n, k: (0, n)),
        ],
        out_specs=pl.BlockSpec((B, NT), lambda n, k: (0, n)),
        out_shape=jax.ShapeDtypeStruct((B, 1024), F32),
        scratch_shapes=[pltpu.VMEM((B, NT), F32)],
        compiler_params=pltpu.CompilerParams(
            dimension_semantics=("parallel", "arbitrary")),
    )(hflat, w1r, row(bfc1))

    return pl.pallas_call(
        _fc2_kernel,
        in_specs=[pl.BlockSpec(z.shape, lambda: (0, 0)),
                  pl.BlockSpec(Wfc2.shape, lambda: (0, 0)),
                  pl.BlockSpec((1, 10), lambda: (0, 0))],
        out_specs=pl.BlockSpec((B, 10), lambda: (0, 0)),
        out_shape=jax.ShapeDtypeStruct((B, 10), F32),
    )(z, Wfc2, row(bfc2))
